# Initial kernel scaffold; baseline (speedup 1.0000x reference)
import jax
import jax.numpy as jnp
from jax.experimental import pallas as pl


def kernel(x):
    raise NotImplementedError("write your pallas kernel here")


# speedup vs baseline: 71.7693x; 71.7693x over previous
import functools
import math

import jax
import jax.numpy as jnp
from jax.experimental import pallas as pl

_NBINS = 9
_POOL = 7
_H = 224
_W = 224
_HC = _H // _POOL
_WC = _W // _POOL

_SIN = tuple(math.sin(k * math.pi / _NBINS) for k in range(_NBINS))
_COS = tuple(math.cos(k * math.pi / _NBINS) for k in range(_NBINS))


def _hog_body(x_ref, o_ref):
    x = x_ref[0, 0].astype(jnp.bfloat16).astype(jnp.float32)

    xm = jnp.concatenate([x[1:2, :], x[:-1, :]], axis=0)
    xp = jnp.concatenate([x[1:, :], x[-2:-1, :]], axis=0)
    s = xm + 2.0 * x + xp
    d = xm - xp

    sl = jnp.concatenate([s[:, 1:2], s[:, :-1]], axis=1)
    sr = jnp.concatenate([s[:, 1:], s[:, -2:-1]], axis=1)
    dl = jnp.concatenate([d[:, 1:2], d[:, :-1]], axis=1)
    dr = jnp.concatenate([d[:, 1:], d[:, -2:-1]], axis=1)
    gx = sl - sr
    gy = dl + 2.0 * d + dr

    norm = jnp.sqrt(gx * gx + gy * gy)

    neg = (gx < 0.0) | ((gx == 0.0) & (gy < 0.0))
    u = jnp.where(neg, -gy, gy)
    v = jnp.where(neg, -gx, gx)

    t = [norm]
    for k in range(1, _NBINS):
        mask = (v * _COS[k] - u * _SIN[k]) >= 0.0
        t.append(jnp.where(mask, norm, 0.0))

    rows = []
    for k in range(_NBINS):
        plane = t[k] - t[k + 1] if k + 1 < _NBINS else t[k]
        rows.append(plane.reshape(_HC, _POOL, _W).sum(axis=1))
    stacked = jnp.concatenate(rows, axis=0)

    j = jax.lax.broadcasted_iota(jnp.int32, (_W, _WC), 0)
    c = jax.lax.broadcasted_iota(jnp.int32, (_W, _WC), 1)
    pool_mat = ((j // _POOL) == c).astype(jnp.float32)
    cells = jnp.dot(stacked, pool_mat, preferred_element_type=jnp.float32,
                    precision=jax.lax.Precision.HIGHEST)

    sq = cells * cells
    acc = sq[0:_HC]
    for k in range(1, _NBINS):
        acc = acc + sq[k * _HC:(k + 1) * _HC]
    inv = 1.0 / jnp.maximum(jnp.sqrt(acc), 1e-12)
    for k in range(_NBINS):
        o_ref[0, 0, k] = cells[k * _HC:(k + 1) * _HC] * inv


@jax.jit
def kernel(x):
    b, c, h, w = x.shape
    return pl.pallas_call(
        _hog_body,
        grid=(b, c),
        in_specs=[pl.BlockSpec((1, 1, h, w), lambda i, j: (i, j, 0, 0))],
        out_specs=pl.BlockSpec((1, 1, _NBINS, _HC, _WC),
                               lambda i, j: (i, j, 0, 0, 0)),
        out_shape=jax.ShapeDtypeStruct((b, c, _NBINS, _HC, _WC), jnp.float32),
    )(x)


# speedup vs baseline: 72.0391x; 1.0038x over previous
import functools
import math

import jax
import jax.numpy as jnp
import numpy as np
from jax.experimental import pallas as pl

_NBINS = 9
_POOL = 7
_H = 224
_W = 224
_HC = _H // _POOL
_WC = _W // _POOL

_SIN = tuple(math.sin(k * math.pi / _NBINS) for k in range(_NBINS))
_COS = tuple(math.cos(k * math.pi / _NBINS) for k in range(_NBINS))


def _band_matrices():
    i = np.arange(_H)[:, None]
    r = np.arange(_H)[None, :]
    a_s = (2.0 * (r == i) + (r == i - 1) + (r == i + 1)).astype(np.float32)
    a_s[0, 1] += 1.0
    a_s[_H - 1, _H - 2] += 1.0
    a_d = ((r == i - 1).astype(np.float32) - (r == i + 1))
    a_d[0, :] = 0.0
    a_d[_H - 1, :] = 0.0
    return jnp.asarray(a_s, jnp.bfloat16), jnp.asarray(a_d, jnp.bfloat16)


def _pool_matrix():
    j = np.arange(_W)[:, None]
    c = np.arange(_WC)[None, :]
    return jnp.asarray((j // _POOL == c).astype(np.float32))


def _hog_body(x_ref, as_ref, ad_ref, pm_ref, o_ref):
    xb = x_ref[0, 0].astype(jnp.bfloat16)

    s = jnp.dot(as_ref[...], xb, preferred_element_type=jnp.float32)
    d = jnp.dot(ad_ref[...], xb, preferred_element_type=jnp.float32)

    sl = jnp.concatenate([s[:, 1:2], s[:, :-1]], axis=1)
    sr = jnp.concatenate([s[:, 1:], s[:, -2:-1]], axis=1)
    dl = jnp.concatenate([d[:, 1:2], d[:, :-1]], axis=1)
    dr = jnp.concatenate([d[:, 1:], d[:, -2:-1]], axis=1)
    gx = sl - sr
    gy = dl + 2.0 * d + dr

    norm = jnp.sqrt(gx * gx + gy * gy)

    neg = (gx < 0.0) | ((gx == 0.0) & (gy < 0.0))
    u = jnp.where(neg, -gy, gy)
    v = jnp.where(neg, -gx, gx)

    rows = [norm.reshape(_HC, _POOL, _W).sum(axis=1)]
    for k in range(1, _NBINS):
        mask = (v * _COS[k] - u * _SIN[k]) >= 0.0
        t_k = jnp.where(mask, norm, 0.0)
        rows.append(t_k.reshape(_HC, _POOL, _W).sum(axis=1))
    stacked = jnp.concatenate(rows, axis=0)

    q = jnp.dot(stacked, pm_ref[...], preferred_element_type=jnp.float32,
                precision=jax.lax.Precision.HIGHEST)

    cells = []
    acc = None
    for k in range(_NBINS):
        ck = q[k * _HC:(k + 1) * _HC]
        if k + 1 < _NBINS:
            ck = ck - q[(k + 1) * _HC:(k + 2) * _HC]
        cells.append(ck)
        sq = ck * ck
        acc = sq if acc is None else acc + sq
    inv = 1.0 / jnp.maximum(jnp.sqrt(acc), 1e-12)
    for k in range(_NBINS):
        o_ref[0, 0, k] = cells[k] * inv


@jax.jit
def kernel(x):
    b, c, h, w = x.shape
    a_s, a_d = _band_matrices()
    pm = _pool_matrix()
    full = lambda i, j: (0, 0)
    return pl.pallas_call(
        _hog_body,
        grid=(b, c),
        in_specs=[
            pl.BlockSpec((1, 1, h, w), lambda i, j: (i, j, 0, 0)),
            pl.BlockSpec((_H, _H), full),
            pl.BlockSpec((_H, _H), full),
            pl.BlockSpec((_W, _WC), full),
        ],
        out_specs=pl.BlockSpec((1, 1, _NBINS, _HC, _WC),
                               lambda i, j: (i, j, 0, 0, 0)),
        out_shape=jax.ShapeDtypeStruct((b, c, _NBINS, _HC, _WC), jnp.float32),
    )(x, a_s, a_d, pm)


# speedup vs baseline: 129.8583x; 1.8026x over previous
import functools
import math

import jax
import jax.numpy as jnp
import numpy as np
from jax.experimental import pallas as pl

_NBINS = 9
_POOL = 7
_H = 224
_W = 224
_HC = _H // _POOL
_WC = _W // _POOL

_SIN = tuple(math.sin(k * math.pi / _NBINS) for k in range(_NBINS))
_COS = tuple(math.cos(k * math.pi / _NBINS) for k in range(_NBINS))


def _band_matrices():
    i = np.arange(_H)[:, None]
    r = np.arange(_H)[None, :]
    a_s = (2.0 * (r == i) + (r == i - 1) + (r == i + 1)).astype(np.float32)
    a_s[0, 1] += 1.0
    a_s[_H - 1, _H - 2] += 1.0
    a_d = ((r == i - 1).astype(np.float32) - (r == i + 1))
    a_d[0, :] = 0.0
    a_d[_H - 1, :] = 0.0
    return jnp.asarray(a_s, jnp.bfloat16), jnp.asarray(a_d, jnp.bfloat16)


def _pool_matrix():
    j = np.arange(_W)[:, None]
    c = np.arange(_WC)[None, :]
    return jnp.asarray((j // _POOL == c).astype(np.float32))


def _hog_body(x_ref, as_ref, ad_ref, pm_ref, pmt_ref, o_ref):
    xb = x_ref[0, 0].astype(jnp.bfloat16)

    s = jnp.dot(as_ref[...], xb, preferred_element_type=jnp.float32)
    d = jnp.dot(ad_ref[...], xb, preferred_element_type=jnp.float32)

    sl = jnp.concatenate([s[:, 1:2], s[:, :-1]], axis=1)
    sr = jnp.concatenate([s[:, 1:], s[:, -2:-1]], axis=1)
    dl = jnp.concatenate([d[:, 1:2], d[:, :-1]], axis=1)
    dr = jnp.concatenate([d[:, 1:], d[:, -2:-1]], axis=1)
    gx = sl - sr
    gy = dl + 2.0 * d + dr

    norm = jnp.sqrt(gx * gx + gy * gy)

    neg = (gx < 0.0) | ((gx == 0.0) & (gy < 0.0))
    u = jnp.where(neg, -gy, gy)
    v = jnp.where(neg, -gx, gx)

    pm = pm_ref[...]
    cols = [jnp.dot(norm, pm, preferred_element_type=jnp.float32)]
    for k in range(1, _NBINS):
        mask = (v * _COS[k] - u * _SIN[k]) >= 0.0
        t_k = jnp.where(mask, norm, 0.0)
        cols.append(jnp.dot(t_k, pm, preferred_element_type=jnp.float32))
    stacked = jnp.concatenate(cols, axis=1)

    q = jnp.dot(pmt_ref[...], stacked,
                preferred_element_type=jnp.float32,
                precision=jax.lax.Precision.HIGHEST)

    cells = []
    acc = None
    for k in range(_NBINS):
        ck = q[:, k * _WC:(k + 1) * _WC]
        if k + 1 < _NBINS:
            ck = ck - q[:, (k + 1) * _WC:(k + 2) * _WC]
        cells.append(ck)
        sq = ck * ck
        acc = sq if acc is None else acc + sq
    inv = 1.0 / jnp.maximum(jnp.sqrt(acc), 1e-12)
    for k in range(_NBINS):
        o_ref[0, 0, k] = cells[k] * inv


@jax.jit
def kernel(x):
    b, c, h, w = x.shape
    a_s, a_d = _band_matrices()
    pm = _pool_matrix()
    full = lambda i, j: (0, 0)
    return pl.pallas_call(
        _hog_body,
        grid=(b, c),
        in_specs=[
            pl.BlockSpec((1, 1, h, w), lambda i, j: (i, j, 0, 0)),
            pl.BlockSpec((_H, _H), full),
            pl.BlockSpec((_H, _H), full),
            pl.BlockSpec((_W, _WC), full),
            pl.BlockSpec((_HC, _H), full),
        ],
        out_specs=pl.BlockSpec((1, 1, _NBINS, _HC, _WC),
                               lambda i, j: (i, j, 0, 0, 0)),
        out_shape=jax.ShapeDtypeStruct((b, c, _NBINS, _HC, _WC), jnp.float32),
    )(x, a_s, a_d, pm, pm.T)


# speedup vs baseline: 152.4908x; 1.1743x over previous
import functools
import math

import jax
import jax.numpy as jnp
import numpy as np
from jax.experimental import pallas as pl

_NBINS = 9
_POOL = 7
_H = 224
_W = 224
_HC = _H // _POOL
_WC = _W // _POOL

_SIN = tuple(math.sin(k * math.pi / _NBINS) for k in range(_NBINS))
_COS = tuple(math.cos(k * math.pi / _NBINS) for k in range(_NBINS))


def _band_matrices():
    i = np.arange(_H)[:, None]
    r = np.arange(_H)[None, :]
    a_s = (2.0 * (r == i) + (r == i - 1) + (r == i + 1)).astype(np.float32)
    a_s[0, 1] += 1.0
    a_s[_H - 1, _H - 2] += 1.0
    a_d = ((r == i - 1).astype(np.float32) - (r == i + 1))
    a_d[0, :] = 0.0
    a_d[_H - 1, :] = 0.0
    return jnp.asarray(a_s, jnp.bfloat16), jnp.asarray(a_d, jnp.bfloat16)


def _pool_matrix():
    j = np.arange(_W)[:, None]
    c = np.arange(_WC)[None, :]
    return jnp.asarray((j // _POOL == c).astype(np.float32))


def _hog_body(x_ref, as_ref, ad_ref, pm_ref, pmt_ref, o_ref):
    for ch in range(x_ref.shape[1]):
        _hog_one(x_ref, as_ref, ad_ref, pm_ref, pmt_ref, o_ref, ch)


def _hog_one(x_ref, as_ref, ad_ref, pm_ref, pmt_ref, o_ref, ch):
    xb = x_ref[0, ch].astype(jnp.bfloat16)

    s = jnp.dot(as_ref[...], xb, preferred_element_type=jnp.float32)
    d = jnp.dot(ad_ref[...], xb, preferred_element_type=jnp.float32)

    sl = jnp.concatenate([s[:, 1:2], s[:, :-1]], axis=1)
    sr = jnp.concatenate([s[:, 1:], s[:, -2:-1]], axis=1)
    dl = jnp.concatenate([d[:, 1:2], d[:, :-1]], axis=1)
    dr = jnp.concatenate([d[:, 1:], d[:, -2:-1]], axis=1)
    gx = sl - sr
    gy = dl + 2.0 * d + dr

    norm = jnp.sqrt(gx * gx + gy * gy)

    neg = (gx < 0.0) | ((gx == 0.0) & (gy < 0.0))
    u = jnp.where(neg, -gy, gy)
    v = jnp.where(neg, -gx, gx)

    pm = pm_ref[...]
    cols = [jnp.dot(norm, pm, preferred_element_type=jnp.float32)]
    for k in range(1, _NBINS):
        mask = (v * _COS[k] - u * _SIN[k]) >= 0.0
        t_k = jnp.where(mask, norm, 0.0)
        cols.append(jnp.dot(t_k, pm, preferred_element_type=jnp.float32))
    stacked = jnp.concatenate(cols, axis=1)

    q = jnp.dot(pmt_ref[...], stacked,
                preferred_element_type=jnp.float32,
                precision=jax.lax.Precision.HIGHEST)

    cells = []
    acc = None
    for k in range(_NBINS):
        ck = q[:, k * _WC:(k + 1) * _WC]
        if k + 1 < _NBINS:
            ck = ck - q[:, (k + 1) * _WC:(k + 2) * _WC]
        cells.append(ck)
        sq = ck * ck
        acc = sq if acc is None else acc + sq
    inv = 1.0 / jnp.maximum(jnp.sqrt(acc), 1e-12)
    for k in range(_NBINS):
        o_ref[0, ch, k] = cells[k] * inv


@jax.jit
def kernel(x):
    b, c, h, w = x.shape
    a_s, a_d = _band_matrices()
    pm = _pool_matrix()
    pmt = pm.T.astype(jnp.float32)
    full = lambda i, j: (0, 0)
    return pl.pallas_call(
        _hog_body,
        grid=(b,),
        in_specs=[
            pl.BlockSpec((1, c, h, w), lambda i: (i, 0, 0, 0)),
            pl.BlockSpec((_H, _H), lambda i: (0, 0)),
            pl.BlockSpec((_H, _H), lambda i: (0, 0)),
            pl.BlockSpec((_W, _WC), lambda i: (0, 0)),
            pl.BlockSpec((_HC, _H), lambda i: (0, 0)),
        ],
        out_specs=pl.BlockSpec((1, c, _NBINS, _HC, _WC),
                               lambda i: (i, 0, 0, 0, 0)),
        out_shape=jax.ShapeDtypeStruct((b, c, _NBINS, _HC, _WC), jnp.float32),
    )(x, a_s, a_d, pm, pmt)


# speedup vs baseline: 167.3488x; 1.0974x over previous
import functools
import math

import jax
import jax.numpy as jnp
import numpy as np
from jax.experimental import pallas as pl

_NBINS = 9
_POOL = 7
_H = 224
_W = 224
_HC = _H // _POOL
_WC = _W // _POOL

_SIN = tuple(math.sin(k * math.pi / _NBINS) for k in range(_NBINS))
_COS = tuple(math.cos(k * math.pi / _NBINS) for k in range(_NBINS))


def _band_matrices():
    i = np.arange(_H)[:, None]
    r = np.arange(_H)[None, :]
    a_s = (2.0 * (r == i) + (r == i - 1) + (r == i + 1)).astype(np.float32)
    a_s[0, 1] += 1.0
    a_s[_H - 1, _H - 2] += 1.0
    a_d = ((r == i - 1).astype(np.float32) - (r == i + 1))
    a_d[0, :] = 0.0
    a_d[_H - 1, :] = 0.0
    return jnp.asarray(a_s, jnp.bfloat16), jnp.asarray(a_d, jnp.bfloat16)


def _pool_matrix():
    j = np.arange(_W)[:, None]
    c = np.arange(_WC)[None, :]
    return jnp.asarray((j // _POOL == c).astype(np.float32))


def _hog_body(x_ref, as_ref, ad_ref, pm_ref, pmt_ref, o_ref):
    qs = [_hog_pool(x_ref, as_ref, ad_ref, pm_ref, pmt_ref, ch)
          for ch in range(x_ref.shape[1])]
    for ch, q in enumerate(qs):
        cells = []
        acc = None
        for k in range(_NBINS):
            ck = q[k * _WC:(k + 1) * _WC]
            if k + 1 < _NBINS:
                ck = ck - q[(k + 1) * _WC:(k + 2) * _WC]
            cells.append(ck)
            sq = ck * ck
            acc = sq if acc is None else acc + sq
        inv = 1.0 / jnp.maximum(jnp.sqrt(acc), 1e-12)
        for k in range(_NBINS):
            o_ref[0, ch, k] = (cells[k] * inv).T


def _hog_pool(x_ref, as_ref, ad_ref, pm_ref, pmt_ref, ch):
    xb = x_ref[0, ch].astype(jnp.bfloat16)

    s = jnp.dot(as_ref[...], xb, preferred_element_type=jnp.float32)
    d = jnp.dot(ad_ref[...], xb, preferred_element_type=jnp.float32)

    sl = jnp.concatenate([s[:, 1:2], s[:, :-1]], axis=1)
    sr = jnp.concatenate([s[:, 1:], s[:, -2:-1]], axis=1)
    dl = jnp.concatenate([d[:, 1:2], d[:, :-1]], axis=1)
    dr = jnp.concatenate([d[:, 1:], d[:, -2:-1]], axis=1)
    gx = sl - sr
    gy = dl + 2.0 * d + dr

    norm = jnp.sqrt(gx * gx + gy * gy)

    neg = (gx < 0.0) | ((gx == 0.0) & (gy < 0.0))
    u = jnp.where(neg, -gy, gy)
    v = jnp.where(neg, -gx, gx)

    pm = pm_ref[...]
    cols = [jnp.dot(norm, pm, preferred_element_type=jnp.float32)]
    for k in range(1, _NBINS):
        mask = (v * _COS[k] - u * _SIN[k]) >= 0.0
        t_k = jnp.where(mask, norm, 0.0)
        cols.append(jnp.dot(t_k, pm, preferred_element_type=jnp.float32))
    stacked = jnp.concatenate(cols, axis=1)

    return jnp.dot(stacked.T, pm_ref[...].astype(jnp.float32),
                   preferred_element_type=jnp.float32,
                   precision=jax.lax.Precision.HIGHEST)


@jax.jit
def kernel(x):
    b, c, h, w = x.shape
    a_s, a_d = _band_matrices()
    pm = _pool_matrix()
    pmt = pm.T.astype(jnp.float32)
    full = lambda i, j: (0, 0)
    return pl.pallas_call(
        _hog_body,
        grid=(b,),
        in_specs=[
            pl.BlockSpec((1, c, h, w), lambda i: (i, 0, 0, 0)),
            pl.BlockSpec((_H, _H), lambda i: (0, 0)),
            pl.BlockSpec((_H, _H), lambda i: (0, 0)),
            pl.BlockSpec((_W, _WC), lambda i: (0, 0)),
            pl.BlockSpec((_HC, _H), lambda i: (0, 0)),
        ],
        out_specs=pl.BlockSpec((1, c, _NBINS, _HC, _WC),
                               lambda i: (i, 0, 0, 0, 0)),
        out_shape=jax.ShapeDtypeStruct((b, c, _NBINS, _HC, _WC), jnp.float32),
    )(x, a_s, a_d, pm, pmt)


# speedup vs baseline: 186.1523x; 1.1124x over previous
import functools
import math

import jax
import jax.numpy as jnp
import numpy as np
from jax.experimental import pallas as pl

_NBINS = 9
_POOL = 7
_H = 224
_W = 224
_HC = _H // _POOL
_WC = _W // _POOL

_SIN = tuple(math.sin(k * math.pi / _NBINS) for k in range(_NBINS))
_COS = tuple(math.cos(k * math.pi / _NBINS) for k in range(_NBINS))


def _band_matrices():
    i = np.arange(_H)[:, None]
    r = np.arange(_H)[None, :]
    a_s = (2.0 * (r == i) + (r == i - 1) + (r == i + 1)).astype(np.float32)
    a_s[0, 1] += 1.0
    a_s[_H - 1, _H - 2] += 1.0
    a_d = ((r == i - 1).astype(np.float32) - (r == i + 1))
    a_d[0, :] = 0.0
    a_d[_H - 1, :] = 0.0
    return jnp.asarray(a_s, jnp.bfloat16), jnp.asarray(a_d, jnp.bfloat16)


def _pool_matrix():
    j = np.arange(_W)[:, None]
    c = np.arange(_WC)[None, :]
    return jnp.asarray((j // _POOL == c).astype(np.float32))


def _hog_body(x_ref, as_ref, ad_ref, pm_ref, pmt_ref, o_ref):
    qs = [_hog_pool(x_ref, as_ref, ad_ref, pm_ref, pmt_ref, ch)
          for ch in range(x_ref.shape[1])]
    for ch, q in enumerate(qs):
        cells = []
        acc = None
        for k in range(_NBINS):
            ck = q[k * _WC:(k + 1) * _WC]
            if k + 1 < _NBINS:
                ck = ck - q[(k + 1) * _WC:(k + 2) * _WC]
            cells.append(ck)
            sq = ck * ck
            acc = sq if acc is None else acc + sq
        inv = jnp.minimum(jax.lax.rsqrt(acc), 1e12)
        for k in range(_NBINS):
            o_ref[0, ch, k] = (cells[k] * inv).T


def _hog_pool(x_ref, as_ref, ad_ref, pm_ref, pmt_ref, ch):
    xb = x_ref[0, ch].astype(jnp.bfloat16)

    s = jnp.dot(as_ref[...], xb, preferred_element_type=jnp.float32)
    d = jnp.dot(ad_ref[...], xb, preferred_element_type=jnp.float32)

    sl = jnp.concatenate([s[:, 1:2], s[:, :-1]], axis=1)
    sr = jnp.concatenate([s[:, 1:], s[:, -2:-1]], axis=1)
    dl = jnp.concatenate([d[:, 1:2], d[:, :-1]], axis=1)
    dr = jnp.concatenate([d[:, 1:], d[:, -2:-1]], axis=1)
    gx = sl - sr
    gy = dl + 2.0 * d + dr

    norm = jnp.sqrt(gx * gx + gy * gy)

    neg = (gx < 0.0) | ((gx == 0.0) & (gy < 0.0))
    u = jnp.where(neg, -gy, gy)
    v = jnp.where(neg, -gx, gx)

    pm = pm_ref[...]
    cols = [jnp.dot(norm, pm, preferred_element_type=jnp.float32).T]
    for k in range(1, _NBINS):
        mask = (v * _COS[k] - u * _SIN[k]) >= 0.0
        t_k = jnp.where(mask, norm, 0.0)
        cols.append(jnp.dot(t_k, pm, preferred_element_type=jnp.float32).T)
    stacked = jnp.concatenate(cols, axis=0)

    return jnp.dot(stacked, pm_ref[...].astype(jnp.float32),
                   preferred_element_type=jnp.float32,
                   precision=jax.lax.Precision.HIGHEST)


@jax.jit
def kernel(x):
    b, c, h, w = x.shape
    a_s, a_d = _band_matrices()
    pm = _pool_matrix()
    pmt = pm.T.astype(jnp.float32)
    full = lambda i, j: (0, 0)
    return pl.pallas_call(
        _hog_body,
        grid=(b,),
        in_specs=[
            pl.BlockSpec((1, c, h, w), lambda i: (i, 0, 0, 0)),
            pl.BlockSpec((_H, _H), lambda i: (0, 0)),
            pl.BlockSpec((_H, _H), lambda i: (0, 0)),
            pl.BlockSpec((_W, _WC), lambda i: (0, 0)),
            pl.BlockSpec((_HC, _H), lambda i: (0, 0)),
        ],
        out_specs=pl.BlockSpec((1, c, _NBINS, _HC, _WC),
                               lambda i: (i, 0, 0, 0, 0)),
        out_shape=jax.ShapeDtypeStruct((b, c, _NBINS, _HC, _WC), jnp.float32),
    )(x, a_s, a_d, pm, pmt)
